# baseline (speedup 1.0000x reference)
import jax
import jax.numpy as jnp
from jax.experimental import pallas as pl
from jax.experimental.pallas import tpu as pltpu

_N = 5
_K = 25
_NC = 20
_T = 50
_H = 64
_W = 64

_INTERPRET = False


def _body(out_ref, tgt_tr_ref, tgt_sm, pri_sm, acc_ref, s_bx1, s_bx2, s_by1, s_by2, s_areab):
    b = pl.program_id(0)

    def plane32(c):
        return out_ref[0, 32 * c:32 * (c + 1), :]

    def packed(c_off):
        return jnp.concatenate(
            [plane32(n * _K + c_off) for n in range(_N)], axis=0)

    TTa = tgt_tr_ref[0]
    zpad = jnp.zeros((1, 64 - _T), jnp.float32)

    def stage(ref, row):
        r64 = jnp.concatenate([row, zpad], axis=1)
        ref[...] = jnp.broadcast_to(jnp.transpose(r64, (1, 0)), (64, 128))

    txa = TTa[1:2, :]
    tya = TTa[2:3, :]
    twa = TTa[3:4, :]
    tha = TTa[4:5, :]
    stage(s_bx1, txa - twa / 2.0)
    stage(s_bx2, txa + twa / 2.0)
    stage(s_by1, tya - tha / 2.0)
    stage(s_by2, tya + tha / 2.0)
    stage(s_areab, 0.375 * (twa * tha))

    lio = jax.lax.broadcasted_iota(jnp.int32, (_N * 32, 128), 1)
    sio = jax.lax.broadcasted_iota(jnp.int32, (_N * 32, 128), 0)
    colf = (lio & 63).astype(jnp.float32)
    rowf = (2 * (sio % 32) + (lio >> 6)).astype(jnp.float32)
    aidx = sio // 32
    pw_pri = jnp.full((_N * 32, 128), pri_sm[0], jnp.float32)
    ph_pri = jnp.full((_N * 32, 128), pri_sm[1], jnp.float32)
    for n in range(1, _N):
        pw_pri = jnp.where(aidx == n, pri_sm[2 * n], pw_pri)
        ph_pri = jnp.where(aidx == n, pri_sm[2 * n + 1], ph_pri)

    X = packed(0)
    Y = packed(1)
    Wc = packed(2)
    Hc = packed(3)
    O = packed(4)
    px = (colf + X) / _W
    py = (rowf + Y) / _H
    pw = pw_pri * jnp.exp(Wc) / _W
    ph = ph_pri * jnp.exp(Hc) / _H
    ax1 = px - pw / 2.0
    ax2 = px + pw / 2.0
    ay1 = py - ph / 2.0
    ay2 = py + ph / 2.0
    thr = 0.375 * (pw * ph)
    obj2 = O * O

    def tbody(t, md):
        bx1 = s_bx1[pl.ds(t, 1), :]
        bx2 = s_bx2[pl.ds(t, 1), :]
        by1 = s_by1[pl.ds(t, 1), :]
        by2 = s_by2[pl.ds(t, 1), :]
        areab = s_areab[pl.ds(t, 1), :]
        iw = jnp.maximum(jnp.minimum(ax2, bx2) - jnp.maximum(ax1, bx1), 0.0)
        ih = jnp.maximum(jnp.minimum(ay2, by2) - jnp.maximum(ay1, by1), 0.0)
        inter = iw * ih
        return jnp.maximum(md, inter - (thr + areab))

    neg = jnp.full((_N * 32, 128), -1.0, jnp.float32)
    md = jax.lax.fori_loop(0, _T, tbody, neg, unroll=5)
    noobj_sum = jnp.sum(jnp.where(md > 0.0, 0.0, obj2))

    TT = tgt_tr_ref[0]
    clsr = TT[0:1, :]
    tx0 = TT[1:2, :]
    ty0 = TT[2:3, :]
    tw0 = TT[3:4, :]
    th0 = TT[4:5, :]
    iv = jnp.clip((tx0 * _W).astype(jnp.int32), 0, _W - 1)
    jv = jnp.clip((ty0 * _H).astype(jnp.int32), 0, _H - 1)

    lio2 = jax.lax.broadcasted_iota(jnp.int32, (128, _T), 0)
    colm = ((lio2 & 63) == iv).astype(jnp.float32) * \
        ((lio2 >> 6) == (jv & 1)).astype(jnp.float32)
    A2 = out_ref[0]
    t1 = jnp.dot(A2, colm, preferred_element_type=jnp.float32)
    rio = jax.lax.broadcasted_iota(jnp.int32, (_N * _K * 32, _T), 0)
    rowm = ((rio % 32) == (jv >> 1)).astype(jnp.float32)
    prod = t1 * rowm

    def ch(c):
        return jnp.sum(prod[32 * c:32 * (c + 1), :], axis=0, keepdims=True)

    best = jnp.zeros((1, _T), jnp.int32)
    bestv = jnp.full((1, _T), -1.0, jnp.float32)
    for n in range(_N):
        pwn = pri_sm[2 * n] * jnp.exp(ch(n * _K + 2)) / _W
        phn = pri_sm[2 * n + 1] * jnp.exp(ch(n * _K + 3)) / _H
        inter = jnp.minimum(tw0, pwn) * jnp.minimum(th0, phn)
        union = tw0 * th0 + pwn * phn - inter
        iou = jnp.where(union > 0.0, inter / jnp.where(union > 0.0, union, 1.0), 0.0)
        m = iou > bestv
        best = jnp.where(m, n, best)
        bestv = jnp.where(m, iou, bestv)

    gs = []
    for c in range(_K):
        v = ch(0 * _K + c)
        for n in range(1, _N):
            v = jnp.where(best == n, ch(n * _K + c), v)
        gs.append(v)
    pbw = jnp.full((1, _T), pri_sm[0], jnp.float32)
    pbh = jnp.full((1, _T), pri_sm[1], jnp.float32)
    for n in range(1, _N):
        pbw = jnp.where(best == n, pri_sm[2 * n], pbw)
        pbh = jnp.where(best == n, pri_sm[2 * n + 1], pbh)

    pw_sel = jnp.where(bestv != 0.0, pbw, 0.0)
    ph_sel = jnp.where(bestv != 0.0, pbh, 0.0)
    ivf = iv.astype(jnp.float32)
    jvf = jv.astype(jnp.float32)
    tx = tx0 * _W - ivf
    ty = ty0 * _H - jvf
    okw = pw_sel > 0.0
    okh = ph_sel > 0.0
    tw_l = jnp.where(okw, jnp.log(jnp.where(okw, tw0 * _W / jnp.where(okw, pw_sel, 1.0), 1.0)), 0.0)
    th_l = jnp.where(okh, jnp.log(jnp.where(okh, th0 * _H / jnp.where(okh, ph_sel, 1.0), 1.0)), 0.0)
    scale = 2.0 - tw0 * th0

    upd2 = jnp.zeros((1, _T), jnp.float32)
    for c, tc in enumerate((tx, ty, tw_l, th_l)):
        d = scale * (tc - gs[c])
        upd2 += d * d
    od = 5.0 * (1.0 - gs[4])
    upd2 += od * od
    ci = jnp.clip(clsr.astype(jnp.int32), 0, _NC - 1)
    for k in range(_NC):
        d = (ci == k).astype(jnp.float32) - gs[5 + k]
        upd2 += d * d

    pxc = (ivf + gs[0]) / _W
    pyc = (jvf + gs[1]) / _H
    pwc = pbw * jnp.exp(gs[2]) / _W
    phc = pbh * jnp.exp(gs[3]) / _H
    cax1 = pxc - pwc / 2.0
    cax2 = pxc + pwc / 2.0
    cay1 = pyc - phc / 2.0
    cay2 = pyc + phc / 2.0
    careaA = 0.375 * (pwc * phc)

    def tT(x):
        return jnp.transpose(x, (1, 0))

    bx1r = tx0 - tw0 / 2.0
    bx2r = tx0 + tw0 / 2.0
    by1r = ty0 - th0 / 2.0
    by2r = ty0 + th0 / 2.0
    areabr = 0.375 * (tw0 * th0)
    iwM = jnp.maximum(jnp.minimum(tT(cax2), bx2r) - jnp.maximum(tT(cax1), bx1r), 0.0)
    ihM = jnp.maximum(jnp.minimum(tT(cay2), by2r) - jnp.maximum(tT(cay1), by1r), 0.0)
    diff = iwM * ihM - (tT(careaA) + areabr)
    ignT = jnp.max(diff, axis=1, keepdims=True) > 0.0
    noobjT = jnp.where(ignT, 0.0, tT(gs[4] * gs[4]))

    key = ((jv * _W + iv) * 8 + best).astype(jnp.float32)
    keyT = tT(key)
    tio_s = jax.lax.broadcasted_iota(jnp.int32, (_T, _T), 0)
    tio_l = jax.lax.broadcasted_iota(jnp.int32, (_T, _T), 1)
    taken = jnp.any((keyT == key) & (tio_l > tio_s), axis=1, keepdims=True)
    validT = tT(((tw0 > 0.0) & (th0 > 0.0)).astype(jnp.float32)) > 0.5
    contrib = jnp.where((~taken) & validT, tT(upd2) - noobjT, 0.0)
    corr = jnp.sum(contrib)

    total = noobj_sum + corr

    @pl.when(b == 0)
    def _init():
        acc_ref[0, 0] = total

    @pl.when(b != 0)
    def _acc():
        acc_ref[0, 0] = acc_ref[0, 0] + total


def kernel(output, target, priors):
    B = output.shape[0]
    out3 = output.reshape(B, _N * _K * _H * _W // 128, 128)
    tgt_tr = jnp.transpose(target, (0, 2, 1))
    total = pl.pallas_call(
        _body,
        grid=(B,),
        in_specs=[
            pl.BlockSpec((1, _N * _K * _H * _W // 128, 128), lambda b: (b, 0, 0)),
            pl.BlockSpec((1, 5, _T), lambda b: (b, 0, 0)),
            pl.BlockSpec(memory_space=pltpu.SMEM),
            pl.BlockSpec(memory_space=pltpu.SMEM),
        ],
        out_specs=pl.BlockSpec(memory_space=pltpu.SMEM),
        out_shape=jax.ShapeDtypeStruct((1, 1), jnp.float32),
        scratch_shapes=[pltpu.VMEM((64, 128), jnp.float32)] * 5,
        interpret=_INTERPRET,
    )(out3, tgt_tr, target, priors)
    return jnp.sqrt(total[0, 0]) ** 2


# speedup vs baseline: 1.2934x; 1.2934x over previous
import jax
import jax.numpy as jnp
from jax.experimental import pallas as pl
from jax.experimental.pallas import tpu as pltpu

_N = 5
_K = 25
_NC = 20
_T = 50
_H = 64
_W = 64

_INTERPRET = False


def _body(out_ref, tgt_tr_ref, tgt_sm, pri_sm, acc_ref, s_bx1, s_bx2, s_by1, s_by2, s_areab):
    b = pl.program_id(0)

    def plane(c):
        return out_ref[0, c]

    def pack(p):
        return jnp.concatenate([p[0:32, :], p[32:64, :]], axis=1)

    def packed(c_off):
        return jnp.concatenate(
            [pack(plane(n * _K + c_off)) for n in range(_N)], axis=0)

    TTa = tgt_tr_ref[0]
    zpad = jnp.zeros((1, 64 - _T), jnp.float32)

    def stage(ref, row):
        r64 = jnp.concatenate([row, zpad], axis=1)
        ref[...] = jnp.broadcast_to(jnp.transpose(r64, (1, 0)), (64, 128))

    txa = TTa[1:2, :]
    tya = TTa[2:3, :]
    twa = TTa[3:4, :]
    tha = TTa[4:5, :]
    stage(s_bx1, txa - twa / 2.0)
    stage(s_bx2, txa + twa / 2.0)
    stage(s_by1, tya - tha / 2.0)
    stage(s_by2, tya + tha / 2.0)
    stage(s_areab, 0.375 * (twa * tha))

    lio = jax.lax.broadcasted_iota(jnp.int32, (_N * 32, 128), 1)
    sio = jax.lax.broadcasted_iota(jnp.int32, (_N * 32, 128), 0)
    colf = (lio & 63).astype(jnp.float32)
    rowf = ((sio % 32) + 32 * (lio >> 6)).astype(jnp.float32)
    aidx = sio // 32
    pw_pri = jnp.full((_N * 32, 128), pri_sm[0], jnp.float32)
    ph_pri = jnp.full((_N * 32, 128), pri_sm[1], jnp.float32)
    for n in range(1, _N):
        pw_pri = jnp.where(aidx == n, pri_sm[2 * n], pw_pri)
        ph_pri = jnp.where(aidx == n, pri_sm[2 * n + 1], ph_pri)

    X = packed(0)
    Y = packed(1)
    Wc = packed(2)
    Hc = packed(3)
    O = packed(4)
    px = (colf + X) / _W
    py = (rowf + Y) / _H
    pw = pw_pri * jnp.exp(Wc) / _W
    ph = ph_pri * jnp.exp(Hc) / _H
    ax1 = px - pw / 2.0
    ax2 = px + pw / 2.0
    ay1 = py - ph / 2.0
    ay2 = py + ph / 2.0
    thr = 0.375 * (pw * ph)
    obj2 = O * O

    def tbody(t, md):
        bx1 = s_bx1[pl.ds(t, 1), :]
        bx2 = s_bx2[pl.ds(t, 1), :]
        by1 = s_by1[pl.ds(t, 1), :]
        by2 = s_by2[pl.ds(t, 1), :]
        areab = s_areab[pl.ds(t, 1), :]
        iw = jnp.maximum(jnp.minimum(ax2, bx2) - jnp.maximum(ax1, bx1), 0.0)
        ih = jnp.maximum(jnp.minimum(ay2, by2) - jnp.maximum(ay1, by1), 0.0)
        inter = iw * ih
        return jnp.maximum(md, inter - (thr + areab))

    neg = jnp.full((_N * 32, 128), -1.0, jnp.float32)
    md = jax.lax.fori_loop(0, _T, tbody, neg, unroll=5)
    noobj_sum = jnp.sum(jnp.where(md > 0.0, 0.0, obj2))

    TT = tgt_tr_ref[0]
    clsr = TT[0:1, :]
    tx0 = TT[1:2, :]
    ty0 = TT[2:3, :]
    tw0 = TT[3:4, :]
    th0 = TT[4:5, :]
    iv = jnp.clip((tx0 * _W).astype(jnp.int32), 0, _W - 1)
    jv = jnp.clip((ty0 * _H).astype(jnp.int32), 0, _H - 1)

    wio = jax.lax.broadcasted_iota(jnp.int32, (_W, _T), 0)
    colm = (wio == iv).astype(jnp.float32)
    rowm = (wio == jv).astype(jnp.float32)
    colparts = []
    for n in range(_N):
        t1 = jax.lax.dot_general(
            out_ref[0, n * _K:(n + 1) * _K], colm, (((2,), (0,)), ((), ())),
            preferred_element_type=jnp.float32)
        colparts.append(jnp.sum(t1 * rowm[None], axis=1))

    def ch(c):
        part = colparts[c // _K]
        cc = c % _K
        return part[cc:cc + 1, :]

    best = jnp.zeros((1, _T), jnp.int32)
    bestv = jnp.full((1, _T), -1.0, jnp.float32)
    for n in range(_N):
        pwn = pri_sm[2 * n] * jnp.exp(ch(n * _K + 2)) / _W
        phn = pri_sm[2 * n + 1] * jnp.exp(ch(n * _K + 3)) / _H
        inter = jnp.minimum(tw0, pwn) * jnp.minimum(th0, phn)
        union = tw0 * th0 + pwn * phn - inter
        iou = jnp.where(union > 0.0, inter / jnp.where(union > 0.0, union, 1.0), 0.0)
        m = iou > bestv
        best = jnp.where(m, n, best)
        bestv = jnp.where(m, iou, bestv)

    gs = []
    for c in range(_K):
        v = ch(0 * _K + c)
        for n in range(1, _N):
            v = jnp.where(best == n, ch(n * _K + c), v)
        gs.append(v)
    pbw = jnp.full((1, _T), pri_sm[0], jnp.float32)
    pbh = jnp.full((1, _T), pri_sm[1], jnp.float32)
    for n in range(1, _N):
        pbw = jnp.where(best == n, pri_sm[2 * n], pbw)
        pbh = jnp.where(best == n, pri_sm[2 * n + 1], pbh)

    pw_sel = jnp.where(bestv != 0.0, pbw, 0.0)
    ph_sel = jnp.where(bestv != 0.0, pbh, 0.0)
    ivf = iv.astype(jnp.float32)
    jvf = jv.astype(jnp.float32)
    tx = tx0 * _W - ivf
    ty = ty0 * _H - jvf
    okw = pw_sel > 0.0
    okh = ph_sel > 0.0
    tw_l = jnp.where(okw, jnp.log(jnp.where(okw, tw0 * _W / jnp.where(okw, pw_sel, 1.0), 1.0)), 0.0)
    th_l = jnp.where(okh, jnp.log(jnp.where(okh, th0 * _H / jnp.where(okh, ph_sel, 1.0), 1.0)), 0.0)
    scale = 2.0 - tw0 * th0

    upd2 = jnp.zeros((1, _T), jnp.float32)
    for c, tc in enumerate((tx, ty, tw_l, th_l)):
        d = scale * (tc - gs[c])
        upd2 += d * d
    od = 5.0 * (1.0 - gs[4])
    upd2 += od * od
    ci = jnp.clip(clsr.astype(jnp.int32), 0, _NC - 1)
    for k in range(_NC):
        d = (ci == k).astype(jnp.float32) - gs[5 + k]
        upd2 += d * d

    pxc = (ivf + gs[0]) / _W
    pyc = (jvf + gs[1]) / _H
    pwc = pbw * jnp.exp(gs[2]) / _W
    phc = pbh * jnp.exp(gs[3]) / _H
    cax1 = pxc - pwc / 2.0
    cax2 = pxc + pwc / 2.0
    cay1 = pyc - phc / 2.0
    cay2 = pyc + phc / 2.0
    careaA = 0.375 * (pwc * phc)

    def tT(x):
        return jnp.transpose(x, (1, 0))

    bx1r = tx0 - tw0 / 2.0
    bx2r = tx0 + tw0 / 2.0
    by1r = ty0 - th0 / 2.0
    by2r = ty0 + th0 / 2.0
    areabr = 0.375 * (tw0 * th0)
    iwM = jnp.maximum(jnp.minimum(tT(cax2), bx2r) - jnp.maximum(tT(cax1), bx1r), 0.0)
    ihM = jnp.maximum(jnp.minimum(tT(cay2), by2r) - jnp.maximum(tT(cay1), by1r), 0.0)
    diff = iwM * ihM - (tT(careaA) + areabr)
    ignT = jnp.max(diff, axis=1, keepdims=True) > 0.0
    noobjT = jnp.where(ignT, 0.0, tT(gs[4] * gs[4]))

    key = ((jv * _W + iv) * 8 + best).astype(jnp.float32)
    keyT = tT(key)
    tio_s = jax.lax.broadcasted_iota(jnp.int32, (_T, _T), 0)
    tio_l = jax.lax.broadcasted_iota(jnp.int32, (_T, _T), 1)
    taken = jnp.any((keyT == key) & (tio_l > tio_s), axis=1, keepdims=True)
    validT = tT(((tw0 > 0.0) & (th0 > 0.0)).astype(jnp.float32)) > 0.5
    contrib = jnp.where((~taken) & validT, tT(upd2) - noobjT, 0.0)
    corr = jnp.sum(contrib)

    total = noobj_sum + corr

    @pl.when(b == 0)
    def _init():
        acc_ref[0, 0] = total

    @pl.when(b != 0)
    def _acc():
        acc_ref[0, 0] = acc_ref[0, 0] + total


def kernel(output, target, priors):
    B = output.shape[0]
    tgt_tr = jnp.transpose(target, (0, 2, 1))
    total = pl.pallas_call(
        _body,
        grid=(B,),
        in_specs=[
            pl.BlockSpec((1, _N * _K, _H, _W), lambda b: (b, 0, 0, 0)),
            pl.BlockSpec((1, 5, _T), lambda b: (b, 0, 0)),
            pl.BlockSpec(memory_space=pltpu.SMEM),
            pl.BlockSpec(memory_space=pltpu.SMEM),
        ],
        out_specs=pl.BlockSpec(memory_space=pltpu.SMEM),
        out_shape=jax.ShapeDtypeStruct((1, 1), jnp.float32),
        scratch_shapes=[pltpu.VMEM((64, 128), jnp.float32)] * 5,
        interpret=_INTERPRET,
    )(output, tgt_tr, target, priors)
    return jnp.sqrt(total[0, 0]) ** 2


# speedup vs baseline: 1.2946x; 1.0009x over previous
import jax
import jax.numpy as jnp
from jax.experimental import pallas as pl
from jax.experimental.pallas import tpu as pltpu

_N = 5
_K = 25
_NC = 20
_T = 50
_H = 64
_W = 64

def _body(out_ref, tgt_tr_ref, tgt_sm, pri_sm, acc_ref, s_bx1, s_bx2, s_by1, s_by2, s_areab):
    b = pl.program_id(0)

    def plane(c):
        return out_ref[0, c]

    def pack(p):
        return jnp.concatenate([p[0:32, :], p[32:64, :]], axis=1)

    def packed(c_off):
        return jnp.concatenate(
            [pack(plane(n * _K + c_off)) for n in range(_N)], axis=0)

    TTa = tgt_tr_ref[0]
    zpad = jnp.zeros((1, 64 - _T), jnp.float32)

    def stage(ref, row):
        r64 = jnp.concatenate([row, zpad], axis=1)
        ref[...] = jnp.broadcast_to(jnp.transpose(r64, (1, 0)), (64, 128))

    txa = TTa[1:2, :]
    tya = TTa[2:3, :]
    twa = TTa[3:4, :]
    tha = TTa[4:5, :]
    stage(s_bx1, txa - twa / 2.0)
    stage(s_bx2, txa + twa / 2.0)
    stage(s_by1, tya - tha / 2.0)
    stage(s_by2, tya + tha / 2.0)
    stage(s_areab, 0.375 * (twa * tha))

    lio = jax.lax.broadcasted_iota(jnp.int32, (_N * 32, 128), 1)
    sio = jax.lax.broadcasted_iota(jnp.int32, (_N * 32, 128), 0)
    colf = (lio & 63).astype(jnp.float32)
    rowf = ((sio % 32) + 32 * (lio >> 6)).astype(jnp.float32)
    aidx = sio // 32
    pw_pri = jnp.full((_N * 32, 128), pri_sm[0], jnp.float32)
    ph_pri = jnp.full((_N * 32, 128), pri_sm[1], jnp.float32)
    for n in range(1, _N):
        pw_pri = jnp.where(aidx == n, pri_sm[2 * n], pw_pri)
        ph_pri = jnp.where(aidx == n, pri_sm[2 * n + 1], ph_pri)

    X = packed(0)
    Y = packed(1)
    Wc = packed(2)
    Hc = packed(3)
    O = packed(4)
    px = (colf + X) / _W
    py = (rowf + Y) / _H
    pw = pw_pri * jnp.exp(Wc) / _W
    ph = ph_pri * jnp.exp(Hc) / _H
    ax1 = px - pw / 2.0
    ax2 = px + pw / 2.0
    ay1 = py - ph / 2.0
    ay2 = py + ph / 2.0
    thr = 0.375 * (pw * ph)
    obj2 = O * O

    def tbody(t, md):
        bx1 = s_bx1[pl.ds(t, 1), :]
        bx2 = s_bx2[pl.ds(t, 1), :]
        by1 = s_by1[pl.ds(t, 1), :]
        by2 = s_by2[pl.ds(t, 1), :]
        areab = s_areab[pl.ds(t, 1), :]
        iw = jnp.maximum(jnp.minimum(ax2, bx2) - jnp.maximum(ax1, bx1), 0.0)
        ih = jnp.maximum(jnp.minimum(ay2, by2) - jnp.maximum(ay1, by1), 0.0)
        inter = iw * ih
        return jnp.maximum(md, inter - (thr + areab))

    neg = jnp.full((_N * 32, 128), -1.0, jnp.float32)
    md = jax.lax.fori_loop(0, _T, tbody, neg, unroll=5)
    noobj_sum = jnp.sum(jnp.where(md > 0.0, 0.0, obj2))

    TT = tgt_tr_ref[0]
    clsr = TT[0:1, :]
    tx0 = TT[1:2, :]
    ty0 = TT[2:3, :]
    tw0 = TT[3:4, :]
    th0 = TT[4:5, :]
    iv = jnp.clip((tx0 * _W).astype(jnp.int32), 0, _W - 1)
    jv = jnp.clip((ty0 * _H).astype(jnp.int32), 0, _H - 1)

    wio = jax.lax.broadcasted_iota(jnp.int32, (_W, _T), 0)
    colm = (wio == iv).astype(jnp.float32)
    rowm = (wio == jv).astype(jnp.float32)
    colparts = []
    for n in range(_N):
        t1 = jax.lax.dot_general(
            out_ref[0, n * _K:(n + 1) * _K], colm, (((2,), (0,)), ((), ())),
            preferred_element_type=jnp.float32)
        colparts.append(jnp.sum(t1 * rowm[None], axis=1))

    def ch(c):
        part = colparts[c // _K]
        cc = c % _K
        return part[cc:cc + 1, :]

    best = jnp.zeros((1, _T), jnp.int32)
    bestv = jnp.full((1, _T), -1.0, jnp.float32)
    for n in range(_N):
        pwn = pri_sm[2 * n] * jnp.exp(ch(n * _K + 2)) / _W
        phn = pri_sm[2 * n + 1] * jnp.exp(ch(n * _K + 3)) / _H
        inter = jnp.minimum(tw0, pwn) * jnp.minimum(th0, phn)
        union = tw0 * th0 + pwn * phn - inter
        iou = jnp.where(union > 0.0, inter / jnp.where(union > 0.0, union, 1.0), 0.0)
        m = iou > bestv
        best = jnp.where(m, n, best)
        bestv = jnp.where(m, iou, bestv)

    gs = []
    for c in range(_K):
        v = ch(0 * _K + c)
        for n in range(1, _N):
            v = jnp.where(best == n, ch(n * _K + c), v)
        gs.append(v)
    pbw = jnp.full((1, _T), pri_sm[0], jnp.float32)
    pbh = jnp.full((1, _T), pri_sm[1], jnp.float32)
    for n in range(1, _N):
        pbw = jnp.where(best == n, pri_sm[2 * n], pbw)
        pbh = jnp.where(best == n, pri_sm[2 * n + 1], pbh)

    pw_sel = jnp.where(bestv != 0.0, pbw, 0.0)
    ph_sel = jnp.where(bestv != 0.0, pbh, 0.0)
    ivf = iv.astype(jnp.float32)
    jvf = jv.astype(jnp.float32)
    tx = tx0 * _W - ivf
    ty = ty0 * _H - jvf
    okw = pw_sel > 0.0
    okh = ph_sel > 0.0
    tw_l = jnp.where(okw, jnp.log(jnp.where(okw, tw0 * _W / jnp.where(okw, pw_sel, 1.0), 1.0)), 0.0)
    th_l = jnp.where(okh, jnp.log(jnp.where(okh, th0 * _H / jnp.where(okh, ph_sel, 1.0), 1.0)), 0.0)
    scale = 2.0 - tw0 * th0

    upd2 = jnp.zeros((1, _T), jnp.float32)
    for c, tc in enumerate((tx, ty, tw_l, th_l)):
        d = scale * (tc - gs[c])
        upd2 += d * d
    od = 5.0 * (1.0 - gs[4])
    upd2 += od * od
    ci = jnp.clip(clsr.astype(jnp.int32), 0, _NC - 1)
    for k in range(_NC):
        d = (ci == k).astype(jnp.float32) - gs[5 + k]
        upd2 += d * d

    pxc = (ivf + gs[0]) / _W
    pyc = (jvf + gs[1]) / _H
    pwc = pbw * jnp.exp(gs[2]) / _W
    phc = pbh * jnp.exp(gs[3]) / _H
    cax1 = pxc - pwc / 2.0
    cax2 = pxc + pwc / 2.0
    cay1 = pyc - phc / 2.0
    cay2 = pyc + phc / 2.0
    careaA = 0.375 * (pwc * phc)

    def tT(x):
        return jnp.transpose(x, (1, 0))

    bx1r = tx0 - tw0 / 2.0
    bx2r = tx0 + tw0 / 2.0
    by1r = ty0 - th0 / 2.0
    by2r = ty0 + th0 / 2.0
    areabr = 0.375 * (tw0 * th0)
    iwM = jnp.maximum(jnp.minimum(tT(cax2), bx2r) - jnp.maximum(tT(cax1), bx1r), 0.0)
    ihM = jnp.maximum(jnp.minimum(tT(cay2), by2r) - jnp.maximum(tT(cay1), by1r), 0.0)
    diff = iwM * ihM - (tT(careaA) + areabr)
    ignT = jnp.max(diff, axis=1, keepdims=True) > 0.0
    noobjT = jnp.where(ignT, 0.0, tT(gs[4] * gs[4]))

    key = ((jv * _W + iv) * 8 + best).astype(jnp.float32)
    keyT = tT(key)
    tio_s = jax.lax.broadcasted_iota(jnp.int32, (_T, _T), 0)
    tio_l = jax.lax.broadcasted_iota(jnp.int32, (_T, _T), 1)
    taken = jnp.any((keyT == key) & (tio_l > tio_s), axis=1, keepdims=True)
    validT = tT(((tw0 > 0.0) & (th0 > 0.0)).astype(jnp.float32)) > 0.5
    contrib = jnp.where((~taken) & validT, tT(upd2) - noobjT, 0.0)
    corr = jnp.sum(contrib)

    total = noobj_sum + corr

    @pl.when(b == 0)
    def _init():
        acc_ref[0, 0] = total

    @pl.when(b != 0)
    def _acc():
        acc_ref[0, 0] = acc_ref[0, 0] + total


def kernel(output, target, priors):
    B = output.shape[0]
    tgt_tr = jnp.transpose(target, (0, 2, 1))
    total = pl.pallas_call(
        _body,
        grid=(B,),
        in_specs=[
            pl.BlockSpec((1, _N * _K, _H, _W), lambda b: (b, 0, 0, 0)),
            pl.BlockSpec((1, 5, _T), lambda b: (b, 0, 0)),
            pl.BlockSpec(memory_space=pltpu.SMEM),
            pl.BlockSpec(memory_space=pltpu.SMEM),
        ],
        out_specs=pl.BlockSpec(memory_space=pltpu.SMEM),
        out_shape=jax.ShapeDtypeStruct((1, 1), jnp.float32),
        scratch_shapes=[pltpu.VMEM((64, 128), jnp.float32)] * 5,
    )(output, tgt_tr, target, priors)
    return jnp.sqrt(total[0, 0]) ** 2
